# baseline (speedup 1.0000x reference)
import jax
import jax.numpy as jnp
from jax.experimental import pallas as pl
from jax.experimental.pallas import tpu as pltpu

_CHUNK = 32768
_NBUF = 4


def _make_schedule(n):
    n_al = n - n % 128
    sizes = []
    if n_al >= 8 * _CHUNK:
        sizes += [8192, 16384]
    rem = n_al - sum(sizes)
    nfull = rem // _CHUNK
    if nfull >= 1 and rem - nfull * _CHUNK < 16384:
        nfull -= 1
    sizes += [_CHUNK] * nfull
    rem = n_al - sum(sizes)
    if rem > 16384:
        a = (rem // 2) // 128 * 128
        sizes += [rem - a, a]
    elif rem:
        sizes.append(rem)
    starts, s = [], 0
    for sz in sizes:
        starts.append(s)
        s += sz
    return list(zip(starts, sizes))


def _make_body(sched, tail):
    n_ch = len(sched)
    tail_start = sched[-1][0] + sched[-1][1] if n_ch else 0

    def body(xt_hbm, w_ref, bt_ref, o_hbm, xbuf, obuf, insem, outsem,
             xtail, otail, tsem):
        def in_copy(i):
            s, sz = sched[i]
            slot = i % _NBUF
            return pltpu.make_async_copy(
                xt_hbm.at[:, pl.ds(s, sz)],
                xbuf.at[slot, :, pl.ds(0, sz)],
                insem.at[slot],
            )

        def out_copy(i):
            s, sz = sched[i]
            slot = i % _NBUF
            return pltpu.make_async_copy(
                obuf.at[slot, :, pl.ds(0, sz)],
                o_hbm.at[:, pl.ds(s, sz)],
                outsem.at[slot],
            )

        if tail:
            tail_in = pltpu.make_async_copy(
                xt_hbm.at[:, pl.ds(tail_start, tail)], xtail, tsem)
            tail_in.start()
        for j in range(min(_NBUF, n_ch)):
            in_copy(j).start()
        for i in range(n_ch):
            slot = i % _NBUF
            sz = sched[i][1]
            in_copy(i).wait()
            if i >= _NBUF:
                out_copy(i - _NBUF).wait()
            obuf[slot, :, pl.ds(0, sz)] = (
                jax.lax.dot_general(
                    w_ref[...], xbuf[slot, :, pl.ds(0, sz)],
                    dimension_numbers=(((0,), (0,)), ((), ())),
                    preferred_element_type=jnp.float32,
                )
                + bt_ref[...]
            )
            out_copy(i).start()
            if i + _NBUF < n_ch:
                in_copy(i + _NBUF).start()
        if tail:
            tail_in.wait()
            otail[...] = (
                jax.lax.dot_general(
                    w_ref[...], xtail[...],
                    dimension_numbers=(((0,), (0,)), ((), ())),
                    preferred_element_type=jnp.float32,
                )
                + bt_ref[...]
            )
            tail_out = pltpu.make_async_copy(
                otail, o_hbm.at[:, pl.ds(tail_start, tail)], tsem)
            tail_out.start()
        for j in range(max(n_ch - _NBUF, 0), n_ch):
            out_copy(j).wait()
        if tail:
            tail_out.wait()

    return body


def kernel(input, kernel, bias):
    n, c_in = input.shape
    c_out = kernel.shape[1]
    xt = input.T
    bt = bias.T
    sched = _make_schedule(n)
    tail = n % 128
    out_t = pl.pallas_call(
        _make_body(sched, tail),
        in_specs=[
            pl.BlockSpec(memory_space=pl.ANY),
            pl.BlockSpec(memory_space=pltpu.VMEM),
            pl.BlockSpec(memory_space=pltpu.VMEM),
        ],
        out_specs=pl.BlockSpec(memory_space=pl.ANY),
        out_shape=jax.ShapeDtypeStruct((c_out, n), jnp.float32),
        scratch_shapes=[
            pltpu.VMEM((_NBUF, c_in, _CHUNK), jnp.float32),
            pltpu.VMEM((_NBUF, c_out, _CHUNK), jnp.float32),
            pltpu.SemaphoreType.DMA((_NBUF,)),
            pltpu.SemaphoreType.DMA((_NBUF,)),
            pltpu.VMEM((c_in, max(tail, 8)), jnp.float32),
            pltpu.VMEM((c_out, max(tail, 8)), jnp.float32),
            pltpu.SemaphoreType.DMA,
        ],
    )(xt, kernel, bt)
    return out_t.T


# speedup vs baseline: 1.0048x; 1.0048x over previous
import jax
import jax.numpy as jnp
from jax.experimental import pallas as pl
from jax.experimental.pallas import tpu as pltpu

_CHUNK = 65536
_NBUF = 3


def _make_schedule(n):
    n_al = n - n % 128
    sizes = []
    if n_al >= 8 * _CHUNK:
        sizes += [8192, 16384]
    rem = n_al - sum(sizes)
    nfull = rem // _CHUNK
    if nfull >= 1 and rem - nfull * _CHUNK < 16384:
        nfull -= 1
    sizes += [_CHUNK] * nfull
    rem = n_al - sum(sizes)
    if rem > 16384:
        a = (rem // 2) // 128 * 128
        sizes += [rem - a, a]
    elif rem:
        sizes.append(rem)
    starts, s = [], 0
    for sz in sizes:
        starts.append(s)
        s += sz
    return list(zip(starts, sizes))


def _make_body(sched, tail):
    n_ch = len(sched)
    tail_start = sched[-1][0] + sched[-1][1] if n_ch else 0

    def body(xt_hbm, w_ref, bt_ref, o_hbm, xbuf, obuf, insem, outsem,
             xtail, otail, tsem):
        def in_copy(i):
            s, sz = sched[i]
            slot = i % _NBUF
            return pltpu.make_async_copy(
                xt_hbm.at[:, pl.ds(s, sz)],
                xbuf.at[slot, :, pl.ds(0, sz)],
                insem.at[slot],
            )

        def out_copy(i):
            s, sz = sched[i]
            slot = i % _NBUF
            return pltpu.make_async_copy(
                obuf.at[slot, :, pl.ds(0, sz)],
                o_hbm.at[:, pl.ds(s, sz)],
                outsem.at[slot],
            )

        if tail:
            tail_in = pltpu.make_async_copy(
                xt_hbm.at[:, pl.ds(tail_start, tail)], xtail, tsem)
            tail_in.start()
        for j in range(min(_NBUF, n_ch)):
            in_copy(j).start()
        for i in range(n_ch):
            slot = i % _NBUF
            sz = sched[i][1]
            in_copy(i).wait()
            if i >= _NBUF:
                out_copy(i - _NBUF).wait()
            obuf[slot, :, pl.ds(0, sz)] = (
                jax.lax.dot_general(
                    w_ref[...], xbuf[slot, :, pl.ds(0, sz)],
                    dimension_numbers=(((0,), (0,)), ((), ())),
                    preferred_element_type=jnp.float32,
                )
                + bt_ref[...]
            )
            out_copy(i).start()
            if i + _NBUF < n_ch:
                in_copy(i + _NBUF).start()
        if tail:
            tail_in.wait()
            otail[...] = (
                jax.lax.dot_general(
                    w_ref[...], xtail[...],
                    dimension_numbers=(((0,), (0,)), ((), ())),
                    preferred_element_type=jnp.float32,
                )
                + bt_ref[...]
            )
            tail_out = pltpu.make_async_copy(
                otail, o_hbm.at[:, pl.ds(tail_start, tail)], tsem)
            tail_out.start()
        for j in range(max(n_ch - _NBUF, 0), n_ch):
            out_copy(j).wait()
        if tail:
            tail_out.wait()

    return body


def kernel(input, kernel, bias):
    n, c_in = input.shape
    c_out = kernel.shape[1]
    xt = input.T
    bt = bias.T
    sched = _make_schedule(n)
    tail = n % 128
    out_t = pl.pallas_call(
        _make_body(sched, tail),
        in_specs=[
            pl.BlockSpec(memory_space=pl.ANY),
            pl.BlockSpec(memory_space=pltpu.VMEM),
            pl.BlockSpec(memory_space=pltpu.VMEM),
        ],
        out_specs=pl.BlockSpec(memory_space=pl.ANY),
        out_shape=jax.ShapeDtypeStruct((c_out, n), jnp.float32),
        scratch_shapes=[
            pltpu.VMEM((_NBUF, c_in, _CHUNK), jnp.float32),
            pltpu.VMEM((_NBUF, c_out, _CHUNK), jnp.float32),
            pltpu.SemaphoreType.DMA((_NBUF,)),
            pltpu.SemaphoreType.DMA((_NBUF,)),
            pltpu.VMEM((c_in, max(tail, 8)), jnp.float32),
            pltpu.VMEM((c_out, max(tail, 8)), jnp.float32),
            pltpu.SemaphoreType.DMA,
        ],
    )(xt, kernel, bt)
    return out_t.T


# speedup vs baseline: 1.0360x; 1.0310x over previous
import jax
import jax.numpy as jnp
from jax.experimental import pallas as pl
from jax.experimental.pallas import tpu as pltpu

_BLOCK_COLS = 114688


def _pointwise_mm_block(xt_ref, w_ref, b_ref, ot_ref):
    ot_ref[...] = (
        jax.lax.dot_general(
            w_ref[...], xt_ref[...],
            dimension_numbers=(((0,), (0,)), ((), ())),
            preferred_element_type=jnp.float32,
        )
        + b_ref[...].T
    )


def kernel(input, kernel, bias):
    n, c_in = input.shape
    c_out = kernel.shape[1]
    xt = input.T
    grid = (pl.cdiv(n, _BLOCK_COLS),)
    out_t = pl.pallas_call(
        _pointwise_mm_block,
        grid=grid,
        in_specs=[
            pl.BlockSpec((c_in, _BLOCK_COLS), lambda i: (0, i)),
            pl.BlockSpec((c_in, c_out), lambda i: (0, 0)),
            pl.BlockSpec((1, c_out), lambda i: (0, 0)),
        ],
        out_specs=pl.BlockSpec((c_out, _BLOCK_COLS), lambda i: (0, i)),
        out_shape=jax.ShapeDtypeStruct((c_out, n), jnp.float32),
        compiler_params=pltpu.CompilerParams(
            dimension_semantics=("parallel",),
        ),
    )(xt, kernel, bias)
    return out_t.T


# speedup vs baseline: 1.0379x; 1.0019x over previous
import jax
import jax.numpy as jnp
from jax.experimental import pallas as pl
from jax.experimental.pallas import tpu as pltpu

_BLOCK_COLS = 117760


def _pointwise_mm_block(xt_ref, w_ref, b_ref, ot_ref):
    ot_ref[...] = (
        jax.lax.dot_general(
            w_ref[...], xt_ref[...],
            dimension_numbers=(((0,), (0,)), ((), ())),
            preferred_element_type=jnp.float32,
        )
        + b_ref[...].T
    )


def kernel(input, kernel, bias):
    n, c_in = input.shape
    c_out = kernel.shape[1]
    xt = input.T
    grid = (pl.cdiv(n, _BLOCK_COLS),)
    out_t = pl.pallas_call(
        _pointwise_mm_block,
        grid=grid,
        in_specs=[
            pl.BlockSpec((c_in, _BLOCK_COLS), lambda i: (0, i)),
            pl.BlockSpec((c_in, c_out), lambda i: (0, 0)),
            pl.BlockSpec((1, c_out), lambda i: (0, 0)),
        ],
        out_specs=pl.BlockSpec((c_out, _BLOCK_COLS), lambda i: (0, i)),
        out_shape=jax.ShapeDtypeStruct((c_out, n), jnp.float32),
        compiler_params=pltpu.CompilerParams(
            dimension_semantics=("parallel",),
        ),
    )(xt, kernel, bias)
    return out_t.T
